# baseline (speedup 1.0000x reference)
import functools

import jax
import jax.numpy as jnp
from jax import lax
from jax.experimental import pallas as pl
from jax.experimental.pallas import tpu as pltpu
from jax.experimental.pallas import tpu_sc as plsc


def _sc_distmult(heads, rels, tails, entity_emb, relation_emb):
    B = heads.shape[0]
    D = entity_emb.shape[1]
    info = plsc.get_sparse_core_info()
    NC, NS, L = info.num_cores, info.num_subcores, info.num_lanes
    NW = NC * NS
    assert B % (8 * NW) == 0 and D % L == 0
    bpw = B // NW
    n_groups = bpw // L

    mesh = plsc.VectorSubcoreMesh(core_axis_name="c", subcore_axis_name="s")

    @functools.partial(
        pl.kernel,
        mesh=mesh,
        compiler_params=pltpu.CompilerParams(
            needs_layout_passes=False, use_tc_tiling_on_sc=False),
        out_type=jax.ShapeDtypeStruct((B,), jnp.float32),
        scratch_types=[
            pltpu.VMEM((bpw,), jnp.int32),
            pltpu.VMEM((bpw,), jnp.int32),
            pltpu.VMEM((bpw,), jnp.int32),
            pltpu.VMEM((bpw, D), jnp.float32),
            pltpu.VMEM((bpw, D), jnp.float32),
            pltpu.VMEM((bpw, D), jnp.float32),
            pltpu.VMEM((bpw,), jnp.float32),
            pltpu.SemaphoreType.DMA,
            pltpu.SemaphoreType.DMA,
            pltpu.SemaphoreType.DMA,
        ],
    )
    def k(heads_hbm, rels_hbm, tails_hbm, ent_hbm, rel_hbm, out_hbm,
          hidx, ridx, tidx, eh, rr, et, oc, sem_h, sem_r, sem_t):
        wid = lax.axis_index("s") * NC + lax.axis_index("c")
        base = wid * bpw
        pltpu.sync_copy(heads_hbm.at[pl.ds(base, bpw)], hidx)
        pltpu.sync_copy(rels_hbm.at[pl.ds(base, bpw)], ridx)
        pltpu.sync_copy(tails_hbm.at[pl.ds(base, bpw)], tidx)
        cp_h = pltpu.async_copy(ent_hbm.at[hidx], eh, sem_h)
        cp_r = pltpu.async_copy(rel_hbm.at[ridx], rr, sem_r)
        cp_t = pltpu.async_copy(ent_hbm.at[tidx], et, sem_t)
        cp_h.wait()
        cp_r.wait()
        cp_t.wait()

        def group_body(g, carry):
            rows = g * L + lax.iota(jnp.int32, L)

            def d_body(d, acc):
                cols = jnp.full((L,), 0, jnp.int32) + d
                a = plsc.load_gather(eh, [rows, cols])
                b = plsc.load_gather(rr, [rows, cols])
                c = plsc.load_gather(et, [rows, cols])
                return acc + a * b * c

            acc = lax.fori_loop(0, D, d_body, jnp.zeros((L,), jnp.float32))
            oc[pl.ds(g * L, L)] = acc
            return carry

        lax.fori_loop(0, n_groups, group_body, 0)
        pltpu.sync_copy(oc, out_hbm.at[pl.ds(base, bpw)])

    return k(heads, rels, tails, entity_emb, relation_emb)


def _mlp_body(f_ref, w1_ref, b1_ref, w2_ref, b2_ref, g_ref, bb_ref,
              ws_ref, bs_ref, o_ref):
    f = f_ref[...]
    h = jnp.dot(f, w1_ref[...], preferred_element_type=jnp.float32) + b1_ref[...]
    h = jnp.maximum(h, 0.0)
    h = jnp.dot(h, w2_ref[...], preferred_element_type=jnp.float32) + b2_ref[...]
    mean = jnp.mean(h, axis=-1, keepdims=True)
    var = jnp.mean((h - mean) ** 2, axis=-1, keepdims=True)
    h = (h - mean) * lax.rsqrt(var + 1e-5) * g_ref[...] + bb_ref[...]
    z = jnp.maximum(h, 0.0)
    o_ref[...] = jnp.dot(z, ws_ref[...], preferred_element_type=jnp.float32) + bs_ref[...]


def _tc_meta(feats, W1, b1, W2, b2, ln_g, ln_b, Ws, bs):
    B, F = feats.shape
    D = W1.shape[1]
    block = 2048
    full = lambda s: pl.BlockSpec(s, lambda i: (0,) * len(s))
    out2 = pl.pallas_call(
        _mlp_body,
        grid=(B // block,),
        in_specs=[
            pl.BlockSpec((block, F), lambda i: (i, 0)),
            full((F, D)), full((D,)), full((D, D)), full((D,)),
            full((D,)), full((D,)), full((D, 1)), full((1,)),
        ],
        out_specs=pl.BlockSpec((block, 1), lambda i: (i, 0)),
        out_shape=jax.ShapeDtypeStruct((B, 1), jnp.float32),
    )(feats, W1, b1, W2, b2, ln_g, ln_b, Ws, bs)
    return out2[:, 0]


def kernel(heads, rels, tails, metapath_feats, entity_emb, relation_emb,
           W1, b1, W2, b2, ln_g, ln_b, Ws, bs):
    heads = heads.astype(jnp.int32)
    rels = rels.astype(jnp.int32)
    tails = tails.astype(jnp.int32)
    distmult = _sc_distmult(heads, rels, tails, entity_emb, relation_emb)
    meta = _tc_meta(metapath_feats, W1, b1, W2, b2, ln_g, ln_b, Ws, bs)
    return distmult + meta
